# Initial kernel scaffold; baseline (speedup 1.0000x reference)
import jax
import jax.numpy as jnp
from jax.experimental import pallas as pl


def kernel(feat, edge_index, review_id, cj, ci, review_table, W_map, b_map, W_prob, W_rscore, W_r1, W_r2, W_r3, W_lin, b_lin):
    raise NotImplementedError("write your pallas kernel here")


# speedup vs baseline: 3.0620x; 3.0620x over previous
import dataclasses
import functools

import jax
import jax.numpy as jnp
from jax import lax
from jax.experimental import pallas as pl
from jax.experimental.pallas import tpu as pltpu
from jax.experimental.pallas import tpu_sc as plsc

N = 10000
E = 160000
D = 256
RD = 128

NC = 2
NS = 16
NW = NC * NS

GC = 256
NCHUNK = E // GC
RANGE = 320
ACCR = RANGE + 8
TRASH = RANGE + 4
SCHUNK = 3200
NSCAN = E // SCHUNK
LMAX = 12288
BATCH = 64

_mesh = plsc.VectorSubcoreMesh(core_axis_name="c", subcore_axis_name="s")

_sc_cp = pltpu.CompilerParams()
if "needs_layout_passes" in pltpu.CompilerParams.__dataclass_fields__:
    _sc_cp = dataclasses.replace(_sc_cp, needs_layout_passes=False)


def _hmap_body(feat_ref, w_ref, b_ref, out_ref):
    h = jnp.dot(feat_ref[...], w_ref[...], preferred_element_type=jnp.float32)
    out_ref[...] = h + b_ref[...]


def _tc_hmap(feat, w_map, b_map2):
    blk = 512
    grid = (N + blk - 1) // blk
    return pl.pallas_call(
        _hmap_body,
        grid=(grid,),
        in_specs=[
            pl.BlockSpec((blk, D), lambda i: (i, 0)),
            pl.BlockSpec((D, D), lambda i: (0, 0)),
            pl.BlockSpec((1, D), lambda i: (0, 0)),
        ],
        out_specs=pl.BlockSpec((blk, D), lambda i: (i, 0)),
        out_shape=jax.ShapeDtypeStruct((N, D), jnp.float32),
    )(feat, w_map, b_map2)


@functools.partial(
    pl.kernel,
    mesh=_mesh,
    compiler_params=_sc_cp,
    out_type=(
        jax.ShapeDtypeStruct((E, RD), jnp.float32),
        jax.ShapeDtypeStruct((E, D), jnp.float32),
        jax.ShapeDtypeStruct((E,), jnp.float32),
    ),
    scratch_types=[
        pltpu.VMEM((GC,), jnp.int32),
        pltpu.VMEM((GC,), jnp.int32),
        pltpu.VMEM((GC, RD), jnp.float32),
        pltpu.VMEM((GC, D), jnp.float32),
        pltpu.VMEM((GC,), jnp.float32),
        pltpu.VMEM((N,), jnp.float32),
    ],
)
def _sc_gather(table_hbm, rid_hbm, src_hbm, h_hbm, cj_hbm,
               rev_out, hsrc_out, cjsrc_out,
               rid_v, src_v, rev_v, hsrc_v, cjsrc_v, cj_v):
    wid = lax.axis_index("s") * NC + lax.axis_index("c")
    pltpu.sync_copy(cj_hbm, cj_v)
    niter = (NCHUNK + NW - 1) // NW

    @pl.loop(0, niter)
    def _(t):
        chunk = t * NW + wid

        @pl.when(chunk < NCHUNK)
        def _():
            base = chunk * GC
            pltpu.sync_copy(rid_hbm.at[pl.ds(base, GC)], rid_v)
            pltpu.sync_copy(table_hbm.at[rid_v], rev_v)
            pltpu.sync_copy(rev_v, rev_out.at[pl.ds(base, GC)])
            pltpu.sync_copy(src_hbm.at[pl.ds(base, GC)], src_v)
            pltpu.sync_copy(h_hbm.at[src_v], hsrc_v)
            pltpu.sync_copy(hsrc_v, hsrc_out.at[pl.ds(base, GC)])
            for j in range(GC // 16):
                sl = pl.ds(j * 16, 16)
                cjsrc_v[sl] = plsc.load_gather(cj_v, [src_v[sl]])
            pltpu.sync_copy(cjsrc_v, cjsrc_out.at[pl.ds(base, GC)])


def _mlp_body(rev_ref, hsrc_ref, cjs_ref, wpr_ref, w1_ref, w2_ref, w3_ref, out_ref):
    rv = rev_ref[...]
    pr = jnp.dot(rv, wpr_ref[...], preferred_element_type=jnp.float32)
    pr = jax.nn.sigmoid(pr)
    pa = pr[:, 0:1]
    ra = pr[:, 1:2]
    g = lambda x: 0.5 * x * (1.0 + lax.erf(x * 0.7071067811865476))
    rf = g(jnp.dot(rv, w1_ref[...], preferred_element_type=jnp.float32))
    rf = g(jnp.dot(rf, w2_ref[...], preferred_element_type=jnp.float32))
    rf = jnp.dot(rf, w3_ref[...], preferred_element_type=jnp.float32)
    out_ref[...] = (hsrc_ref[...] * pa + rf * ra) * cjs_ref[...]


def _tc_mlp(rev, hsrc, cjs2, wpr, w1, w2, w3):
    blk = 1280
    grid = E // blk
    return pl.pallas_call(
        _mlp_body,
        grid=(grid,),
        in_specs=[
            pl.BlockSpec((blk, RD), lambda i: (i, 0)),
            pl.BlockSpec((blk, D), lambda i: (i, 0)),
            pl.BlockSpec((blk, 1), lambda i: (i, 0)),
            pl.BlockSpec((RD, 8), lambda i: (0, 0)),
            pl.BlockSpec((RD, D), lambda i: (0, 0)),
            pl.BlockSpec((D, D), lambda i: (0, 0)),
            pl.BlockSpec((D, D), lambda i: (0, 0)),
        ],
        out_specs=pl.BlockSpec((blk, D), lambda i: (i, 0)),
        out_shape=jax.ShapeDtypeStruct((E, D), jnp.float32),
    )(rev, hsrc, cjs2, wpr, w1, w2, w3)


@functools.partial(
    pl.kernel,
    mesh=_mesh,
    compiler_params=_sc_cp,
    out_type=jax.ShapeDtypeStruct((N, D), jnp.float32),
    scratch_types=[
        pltpu.VMEM((ACCR, D), jnp.float32),
        pltpu.VMEM((SCHUNK,), jnp.int32),
        pltpu.VMEM((LMAX + 64,), jnp.int32),
        pltpu.VMEM((BATCH,), jnp.int32),
        pltpu.VMEM((BATCH,), jnp.int32),
        pltpu.VMEM((BATCH, D), jnp.float32),
    ],
)
def _sc_scatter(m_hbm, dst_hbm, zero_hbm, out_hbm,
                acc_v, dstc_v, list_v, eid_v, ldst_v, rows_v):
    wid = lax.axis_index("s") * NC + lax.axis_index("c")
    node_base = wid * RANGE
    iota = lax.iota(jnp.int32, 16)

    pltpu.sync_copy(zero_hbm, acc_v.at[pl.ds(0, 128)])
    pltpu.sync_copy(zero_hbm, acc_v.at[pl.ds(128, 128)])
    pltpu.sync_copy(zero_hbm.at[pl.ds(0, ACCR - 256)], acc_v.at[pl.ds(256, ACCR - 256)])

    def accum_batch(b, _):
        flo = b * BATCH
        for g in range(BATCH // 16):
            sl = pl.ds(g * 16, 16)
            p = list_v[pl.ds(flo + g * 16, 16)]
            eid_v[sl] = lax.shift_right_logical(p, 9)
            ldst_v[sl] = p & 511
        pltpu.sync_copy(m_hbm.at[eid_v], rows_v)

        def edge_body(i, _):
            i_splat = jnp.full((16,), 0, jnp.int32) + i
            r_splat = plsc.load_gather(ldst_v, [i_splat])
            for j in range(D // 16):
                col = iota + j * 16
                v = plsc.load_gather(rows_v, [i_splat, col])
                plsc.addupdate_scatter(acc_v, [r_splat, col], v)
            return 0

        lax.fori_loop(0, BATCH, edge_body, 0)
        return 0

    def scan_chunk(c, cnt):
        pltpu.sync_copy(dst_hbm.at[pl.ds(c * SCHUNK, SCHUNK)], dstc_v)

        def group_body(j, cnt):
            d = dstc_v[pl.ds(j * 16, 16)]
            rel = d - node_base
            ok = (rel >= 0) & (rel < RANGE)
            eid = (c * SCHUNK + j * 16) + iota
            packed = lax.shift_left(eid, 9) | (rel & 511)
            plsc.store_compressed(list_v.at[pl.ds(cnt, 16)], packed, mask=ok)
            return cnt + jnp.sum(ok.astype(jnp.int32))

        cnt = lax.fori_loop(0, SCHUNK // 16, group_body, cnt)
        nb = jnp.where(cnt >= LMAX - SCHUNK, cnt >> 6, 0)
        lax.fori_loop(0, nb, accum_batch, 0)
        for g in range(4):
            v = list_v[pl.ds(nb * BATCH + g * 16, 16)]
            list_v[pl.ds(g * 16, 16)] = v
        return cnt - nb * BATCH

    cnt = lax.fori_loop(0, NSCAN, scan_chunk, 0)

    for g in range(BATCH // 16):
        list_v[pl.ds(cnt + g * 16, 16)] = jnp.full((16,), TRASH, jnp.int32)
    nb = (cnt + BATCH - 1) >> 6
    lax.fori_loop(0, nb, accum_batch, 0)

    @pl.when(wid < NW - 1)
    def _():
        pltpu.sync_copy(acc_v.at[pl.ds(0, 128)], out_hbm.at[pl.ds(node_base, 128)])
        pltpu.sync_copy(acc_v.at[pl.ds(128, 128)], out_hbm.at[pl.ds(node_base + 128, 128)])
        pltpu.sync_copy(acc_v.at[pl.ds(256, 64)], out_hbm.at[pl.ds(node_base + 256, 64)])

    @pl.when(wid == NW - 1)
    def _():
        pltpu.sync_copy(acc_v.at[pl.ds(0, 80)], out_hbm.at[pl.ds(node_base, 80)])


def _final_body(hd_ref, ci_ref, w_ref, b_ref, out_ref):
    x = hd_ref[...] * ci_ref[...]
    out_ref[...] = jnp.dot(x, w_ref[...], preferred_element_type=jnp.float32) + b_ref[...]


def _tc_final(h_dst, ci, w_lin, b_lin2):
    blk = 512
    grid = (N + blk - 1) // blk
    return pl.pallas_call(
        _final_body,
        grid=(grid,),
        in_specs=[
            pl.BlockSpec((blk, D), lambda i: (i, 0)),
            pl.BlockSpec((blk, 1), lambda i: (i, 0)),
            pl.BlockSpec((D, D), lambda i: (0, 0)),
            pl.BlockSpec((1, D), lambda i: (0, 0)),
        ],
        out_specs=pl.BlockSpec((blk, D), lambda i: (i, 0)),
        out_shape=jax.ShapeDtypeStruct((N, D), jnp.float32),
    )(h_dst, ci, w_lin, b_lin2)


def kernel(feat, edge_index, review_id, cj, ci, review_table,
           W_map, b_map, W_prob, W_rscore, W_r1, W_r2, W_r3, W_lin, b_lin):
    src = edge_index[0]
    dst = edge_index[1]
    b_map2 = b_map.reshape(1, D)
    b_lin2 = b_lin.reshape(1, D)
    wpr = jnp.concatenate(
        [W_prob, W_rscore, jnp.zeros((RD, 6), jnp.float32)], axis=1)
    zero_blk = jnp.zeros((128, D), jnp.float32)

    h = _tc_hmap(feat, W_map, b_map2)
    rev, hsrc, cj_src = _sc_gather(review_table, review_id, src, h,
                                   cj.reshape(N))
    m = _tc_mlp(rev, hsrc, cj_src.reshape(E, 1), wpr, W_r1, W_r2, W_r3)
    h_dst = _sc_scatter(m, dst, zero_blk)
    return _tc_final(h_dst, ci, W_lin, b_lin2)


# speedup vs baseline: 3.4238x; 1.1182x over previous
import dataclasses
import functools

import jax
import jax.numpy as jnp
from jax import lax
from jax.experimental import pallas as pl
from jax.experimental.pallas import tpu as pltpu
from jax.experimental.pallas import tpu_sc as plsc

N = 10000
E = 160000
D = 256
RD = 128

NC = 2
NS = 16
NW = NC * NS

GC = 256
NCHUNK = E // GC
RANGE = 320
ACCR = RANGE + 8
TRASH = RANGE + 4
SCHUNK = 2000
NSCAN = E // SCHUNK
LMAX = 6144
BATCH = 64

_mesh = plsc.VectorSubcoreMesh(core_axis_name="c", subcore_axis_name="s")

_sc_cp = pltpu.CompilerParams()
if "needs_layout_passes" in pltpu.CompilerParams.__dataclass_fields__:
    _sc_cp = dataclasses.replace(_sc_cp, needs_layout_passes=False)


def _hmap_body(feat_ref, w_ref, b_ref, out_ref):
    h = jnp.dot(feat_ref[...], w_ref[...], preferred_element_type=jnp.float32)
    out_ref[...] = h + b_ref[...]


def _tc_hmap(feat, w_map, b_map2):
    blk = 512
    grid = (N + blk - 1) // blk
    return pl.pallas_call(
        _hmap_body,
        grid=(grid,),
        in_specs=[
            pl.BlockSpec((blk, D), lambda i: (i, 0)),
            pl.BlockSpec((D, D), lambda i: (0, 0)),
            pl.BlockSpec((1, D), lambda i: (0, 0)),
        ],
        out_specs=pl.BlockSpec((blk, D), lambda i: (i, 0)),
        out_shape=jax.ShapeDtypeStruct((N, D), jnp.float32),
    )(feat, w_map, b_map2)


@functools.partial(
    pl.kernel,
    mesh=_mesh,
    compiler_params=_sc_cp,
    out_type=(
        jax.ShapeDtypeStruct((E, RD), jnp.float32),
        jax.ShapeDtypeStruct((E, D), jnp.float32),
        jax.ShapeDtypeStruct((E,), jnp.float32),
    ),
    scratch_types=[
        pltpu.VMEM((GC,), jnp.int32),
        pltpu.VMEM((GC,), jnp.int32),
        pltpu.VMEM((GC, RD), jnp.float32),
        pltpu.VMEM((GC, D), jnp.float32),
        pltpu.VMEM((GC,), jnp.float32),
        pltpu.VMEM((N,), jnp.float32),
    ],
)
def _sc_gather(table_hbm, rid_hbm, src_hbm, h_hbm, cj_hbm,
               rev_out, hsrc_out, cjsrc_out,
               rid_v, src_v, rev_v, hsrc_v, cjsrc_v, cj_v):
    wid = lax.axis_index("s") * NC + lax.axis_index("c")
    pltpu.sync_copy(cj_hbm, cj_v)
    niter = (NCHUNK + NW - 1) // NW

    @pl.loop(0, niter)
    def _(t):
        chunk = t * NW + wid

        @pl.when(chunk < NCHUNK)
        def _():
            base = chunk * GC
            pltpu.sync_copy(rid_hbm.at[pl.ds(base, GC)], rid_v)
            pltpu.sync_copy(table_hbm.at[rid_v], rev_v)
            pltpu.sync_copy(rev_v, rev_out.at[pl.ds(base, GC)])
            pltpu.sync_copy(src_hbm.at[pl.ds(base, GC)], src_v)
            pltpu.sync_copy(h_hbm.at[src_v], hsrc_v)
            pltpu.sync_copy(hsrc_v, hsrc_out.at[pl.ds(base, GC)])
            for j in range(GC // 16):
                sl = pl.ds(j * 16, 16)
                cjsrc_v[sl] = plsc.load_gather(cj_v, [src_v[sl]])
            pltpu.sync_copy(cjsrc_v, cjsrc_out.at[pl.ds(base, GC)])


def _mlp_body(rev_ref, hsrc_ref, cjs_ref, wpr_ref, w1_ref, w2_ref, w3_ref, out_ref):
    bf = jnp.bfloat16
    rv = rev_ref[...].astype(bf)
    pr = jnp.dot(rv, wpr_ref[...].astype(bf), preferred_element_type=jnp.float32)
    pr = jax.nn.sigmoid(pr)
    pa = pr[:, 0:1]
    ra = pr[:, 1:2]
    g = lambda x: 0.5 * x * (1.0 + lax.erf(x * 0.7071067811865476))
    rf = g(jnp.dot(rv, w1_ref[...].astype(bf), preferred_element_type=jnp.float32))
    rf = g(jnp.dot(rf.astype(bf), w2_ref[...].astype(bf), preferred_element_type=jnp.float32))
    rf = jnp.dot(rf.astype(bf), w3_ref[...].astype(bf), preferred_element_type=jnp.float32)
    out_ref[...] = (hsrc_ref[...] * pa + rf * ra) * cjs_ref[...]


def _tc_mlp(rev, hsrc, cjs2, wpr, w1, w2, w3):
    blk = 1280
    grid = E // blk
    return pl.pallas_call(
        _mlp_body,
        grid=(grid,),
        in_specs=[
            pl.BlockSpec((blk, RD), lambda i: (i, 0)),
            pl.BlockSpec((blk, D), lambda i: (i, 0)),
            pl.BlockSpec((blk, 1), lambda i: (i, 0)),
            pl.BlockSpec((RD, 8), lambda i: (0, 0)),
            pl.BlockSpec((RD, D), lambda i: (0, 0)),
            pl.BlockSpec((D, D), lambda i: (0, 0)),
            pl.BlockSpec((D, D), lambda i: (0, 0)),
        ],
        out_specs=pl.BlockSpec((blk, D), lambda i: (i, 0)),
        out_shape=jax.ShapeDtypeStruct((E, D), jnp.float32),
    )(rev, hsrc, cjs2, wpr, w1, w2, w3)


@functools.partial(
    pl.kernel,
    mesh=_mesh,
    compiler_params=_sc_cp,
    out_type=jax.ShapeDtypeStruct((N, D), jnp.float32),
    scratch_types=[
        pltpu.VMEM((ACCR, D), jnp.float32),
        pltpu.VMEM((SCHUNK,), jnp.int32),
        pltpu.VMEM((SCHUNK,), jnp.int32),
        pltpu.VMEM((LMAX + 64,), jnp.int32),
        pltpu.VMEM((BATCH,), jnp.int32),
        pltpu.VMEM((BATCH,), jnp.int32),
        pltpu.VMEM((BATCH,), jnp.int32),
        pltpu.VMEM((BATCH,), jnp.int32),
        pltpu.VMEM((BATCH, D), jnp.float32),
        pltpu.VMEM((BATCH, D), jnp.float32),
        pltpu.SemaphoreType.DMA,
        pltpu.SemaphoreType.DMA,
        pltpu.SemaphoreType.DMA,
        pltpu.SemaphoreType.DMA,
    ],
)
def _sc_scatter(m_hbm, dst_hbm, zero_hbm, out_hbm,
                acc_v, dstc0, dstc1, list_v, eid0, ldst0, eid1, ldst1,
                rows0, rows1, semr0, semr1, semd0, semd1):
    wid = lax.axis_index("s") * NC + lax.axis_index("c")
    node_base = wid * RANGE
    iota = lax.iota(jnp.int32, 16)

    pltpu.sync_copy(zero_hbm, acc_v.at[pl.ds(0, 128)])
    pltpu.sync_copy(zero_hbm, acc_v.at[pl.ds(128, 128)])
    pltpu.sync_copy(zero_hbm.at[pl.ds(0, ACCR - 256)], acc_v.at[pl.ds(256, ACCR - 256)])

    def unpack(b, eid_r, ldst_r):
        flo = b * BATCH
        for g in range(BATCH // 16):
            sl = pl.ds(g * 16, 16)
            p = list_v[pl.ds(flo + g * 16, 16)]
            eid_r[sl] = lax.shift_right_logical(p, 9)
            ldst_r[sl] = p & 511

    def accum(ldst_r, rows_r):
        def edge_body(i, _):
            i_splat = jnp.zeros((16,), jnp.int32) + i
            r_splat = plsc.load_gather(ldst_r, [i_splat])
            for j in range(D // 16):
                col = iota + j * 16
                v = plsc.load_gather(rows_r, [i_splat, col])
                plsc.addupdate_scatter(acc_v, [r_splat, col], v)
            return 0

        lax.fori_loop(0, BATCH, edge_body, 0)

    def drain(nb):
        @pl.when(nb > 0)
        def _():
            unpack(0, eid0, ldst0)
            pltpu.async_copy(m_hbm.at[eid0], rows0, semr0)

        def pair_body(p, _):
            b1 = 2 * p + 1

            @pl.when(b1 < nb)
            def _():
                unpack(b1, eid1, ldst1)
                pltpu.async_copy(m_hbm.at[eid1], rows1, semr1)

            pltpu.make_async_copy(m_hbm.at[eid0], rows0, semr0).wait()
            accum(ldst0, rows0)

            @pl.when(b1 + 1 < nb)
            def _():
                unpack(b1 + 1, eid0, ldst0)
                pltpu.async_copy(m_hbm.at[eid0], rows0, semr0)

            @pl.when(b1 < nb)
            def _():
                pltpu.make_async_copy(m_hbm.at[eid1], rows1, semr1).wait()
                accum(ldst1, rows1)

            return 0

        lax.fori_loop(0, (nb + 1) >> 1, pair_body, 0)

    def scan_groups(c, dstc_v, cnt):
        def group_body(j, cnt):
            d = dstc_v[pl.ds(j * 16, 16)]
            rel = d - node_base
            ok = (rel >= 0) & (rel < RANGE)
            eid = (c * SCHUNK + j * 16) + iota
            packed = lax.shift_left(eid, 9) | (rel & 511)
            plsc.store_compressed(list_v.at[pl.ds(cnt, 16)], packed, mask=ok)
            return cnt + jnp.sum(ok.astype(jnp.int32))

        cnt = lax.fori_loop(0, SCHUNK // 16, group_body, cnt)
        nb = jnp.where(cnt >= LMAX - SCHUNK, cnt >> 6, 0)
        drain(nb)
        for g in range(4):
            v = list_v[pl.ds(nb * BATCH + g * 16, 16)]
            list_v[pl.ds(g * 16, 16)] = v
        return cnt - nb * BATCH

    pltpu.async_copy(dst_hbm.at[pl.ds(0, SCHUNK)], dstc0, semd0)

    def chunk_pair(p, cnt):
        c0 = 2 * p
        pltpu.async_copy(
            dst_hbm.at[pl.ds((c0 + 1) * SCHUNK, SCHUNK)], dstc1, semd1)
        pltpu.make_async_copy(dst_hbm.at[pl.ds(0, SCHUNK)], dstc0, semd0).wait()
        cnt = scan_groups(c0, dstc0, cnt)

        @pl.when(c0 + 2 < NSCAN)
        def _():
            pltpu.async_copy(
                dst_hbm.at[pl.ds((c0 + 2) * SCHUNK, SCHUNK)], dstc0, semd0)

        pltpu.make_async_copy(dst_hbm.at[pl.ds(0, SCHUNK)], dstc1, semd1).wait()
        cnt = scan_groups(c0 + 1, dstc1, cnt)
        return cnt

    cnt = lax.fori_loop(0, NSCAN // 2, chunk_pair, 0)

    for g in range(BATCH // 16):
        list_v[pl.ds(cnt + g * 16, 16)] = jnp.full((16,), TRASH, jnp.int32)
    drain((cnt + BATCH) >> 6)

    @pl.when(wid < NW - 1)
    def _():
        pltpu.sync_copy(acc_v.at[pl.ds(0, 128)], out_hbm.at[pl.ds(node_base, 128)])
        pltpu.sync_copy(acc_v.at[pl.ds(128, 128)], out_hbm.at[pl.ds(node_base + 128, 128)])
        pltpu.sync_copy(acc_v.at[pl.ds(256, 64)], out_hbm.at[pl.ds(node_base + 256, 64)])

    @pl.when(wid == NW - 1)
    def _():
        pltpu.sync_copy(acc_v.at[pl.ds(0, 80)], out_hbm.at[pl.ds(node_base, 80)])


def _final_body(hd_ref, ci_ref, w_ref, b_ref, out_ref):
    x = hd_ref[...] * ci_ref[...]
    out_ref[...] = jnp.dot(x, w_ref[...], preferred_element_type=jnp.float32) + b_ref[...]


def _tc_final(h_dst, ci, w_lin, b_lin2):
    blk = 512
    grid = (N + blk - 1) // blk
    return pl.pallas_call(
        _final_body,
        grid=(grid,),
        in_specs=[
            pl.BlockSpec((blk, D), lambda i: (i, 0)),
            pl.BlockSpec((blk, 1), lambda i: (i, 0)),
            pl.BlockSpec((D, D), lambda i: (0, 0)),
            pl.BlockSpec((1, D), lambda i: (0, 0)),
        ],
        out_specs=pl.BlockSpec((blk, D), lambda i: (i, 0)),
        out_shape=jax.ShapeDtypeStruct((N, D), jnp.float32),
    )(h_dst, ci, w_lin, b_lin2)


def kernel(feat, edge_index, review_id, cj, ci, review_table,
           W_map, b_map, W_prob, W_rscore, W_r1, W_r2, W_r3, W_lin, b_lin):
    src = edge_index[0]
    dst = edge_index[1]
    b_map2 = b_map.reshape(1, D)
    b_lin2 = b_lin.reshape(1, D)
    wpr = jnp.concatenate(
        [W_prob, W_rscore, jnp.zeros((RD, 6), jnp.float32)], axis=1)
    zero_blk = jnp.zeros((128, D), jnp.float32)

    h = _tc_hmap(feat, W_map, b_map2)
    rev, hsrc, cj_src = _sc_gather(review_table, review_id, src, h,
                                   cj.reshape(N))
    m = _tc_mlp(rev, hsrc, cj_src.reshape(E, 1), wpr, W_r1, W_r2, W_r3)
    h_dst = _sc_scatter(m, dst, zero_blk)
    return _tc_final(h_dst, ci, W_lin, b_lin2)


# speedup vs baseline: 3.5611x; 1.0401x over previous
import dataclasses
import functools

import jax
import jax.numpy as jnp
from jax import lax
from jax.experimental import pallas as pl
from jax.experimental.pallas import tpu as pltpu
from jax.experimental.pallas import tpu_sc as plsc

N = 10000
E = 160000
D = 256
RD = 128

NC = 2
NS = 16
NW = NC * NS

GC = 256
NCHUNK = E // GC
RANGE = 320
ACCR = RANGE + 8
TRASH = RANGE + 4
SCHUNK = 2000
NSCAN = E // SCHUNK
LMAX = 6144
BATCH = 64

_mesh = plsc.VectorSubcoreMesh(core_axis_name="c", subcore_axis_name="s")

_sc_cp = pltpu.CompilerParams()
if "needs_layout_passes" in pltpu.CompilerParams.__dataclass_fields__:
    _sc_cp = dataclasses.replace(_sc_cp, needs_layout_passes=False)


def _hmap_body(feat_ref, w_ref, b_ref, out_ref):
    h = jnp.dot(feat_ref[...], w_ref[...], preferred_element_type=jnp.float32)
    out_ref[...] = h + b_ref[...]


def _tc_hmap(feat, w_map, b_map2):
    blk = 512
    grid = (N + blk - 1) // blk
    return pl.pallas_call(
        _hmap_body,
        grid=(grid,),
        in_specs=[
            pl.BlockSpec((blk, D), lambda i: (i, 0)),
            pl.BlockSpec((D, D), lambda i: (0, 0)),
            pl.BlockSpec((1, D), lambda i: (0, 0)),
        ],
        out_specs=pl.BlockSpec((blk, D), lambda i: (i, 0)),
        out_shape=jax.ShapeDtypeStruct((N, D), jnp.float32),
    )(feat, w_map, b_map2)


@functools.partial(
    pl.kernel,
    mesh=_mesh,
    compiler_params=_sc_cp,
    out_type=(
        jax.ShapeDtypeStruct((E, RD), jnp.float32),
        jax.ShapeDtypeStruct((E, D), jnp.float32),
        jax.ShapeDtypeStruct((E,), jnp.float32),
    ),
    scratch_types=[
        pltpu.VMEM((GC,), jnp.int32),
        pltpu.VMEM((GC,), jnp.int32),
        pltpu.VMEM((GC, RD), jnp.float32),
        pltpu.VMEM((GC, D), jnp.float32),
        pltpu.VMEM((GC,), jnp.float32),
        pltpu.VMEM((N,), jnp.float32),
    ],
)
def _sc_gather(table_hbm, rid_hbm, src_hbm, h_hbm, cj_hbm,
               rev_out, hsrc_out, cjsrc_out,
               rid_v, src_v, rev_v, hsrc_v, cjsrc_v, cj_v):
    wid = lax.axis_index("s") * NC + lax.axis_index("c")
    pltpu.sync_copy(cj_hbm, cj_v)
    niter = (NCHUNK + NW - 1) // NW

    @pl.loop(0, niter)
    def _(t):
        chunk = t * NW + wid

        @pl.when(chunk < NCHUNK)
        def _():
            base = chunk * GC
            pltpu.sync_copy(rid_hbm.at[pl.ds(base, GC)], rid_v)
            pltpu.sync_copy(table_hbm.at[rid_v], rev_v)
            pltpu.sync_copy(rev_v, rev_out.at[pl.ds(base, GC)])
            pltpu.sync_copy(src_hbm.at[pl.ds(base, GC)], src_v)
            pltpu.sync_copy(h_hbm.at[src_v], hsrc_v)
            pltpu.sync_copy(hsrc_v, hsrc_out.at[pl.ds(base, GC)])
            for j in range(GC // 16):
                sl = pl.ds(j * 16, 16)
                cjsrc_v[sl] = plsc.load_gather(cj_v, [src_v[sl]])
            pltpu.sync_copy(cjsrc_v, cjsrc_out.at[pl.ds(base, GC)])


def _mlp_body(rev_ref, hsrc_ref, cjs_ref, wpr_ref, w1_ref, w2_ref, w3_ref, out_ref):
    bf = jnp.bfloat16
    rv = rev_ref[...].astype(bf)
    pr = jnp.dot(rv, wpr_ref[...].astype(bf), preferred_element_type=jnp.float32)
    pr = jax.nn.sigmoid(pr)
    pa = pr[:, 0:1]
    ra = pr[:, 1:2]
    g = lambda x: 0.5 * x * (1.0 + lax.erf(x * 0.7071067811865476))
    rf = g(jnp.dot(rv, w1_ref[...].astype(bf), preferred_element_type=jnp.float32))
    rf = g(jnp.dot(rf.astype(bf), w2_ref[...].astype(bf), preferred_element_type=jnp.float32))
    rf = jnp.dot(rf.astype(bf), w3_ref[...].astype(bf), preferred_element_type=jnp.float32)
    out_ref[...] = (hsrc_ref[...] * pa + rf * ra) * cjs_ref[...]


def _tc_mlp(rev, hsrc, cjs2, wpr, w1, w2, w3):
    blk = 1280
    grid = E // blk
    return pl.pallas_call(
        _mlp_body,
        grid=(grid,),
        in_specs=[
            pl.BlockSpec((blk, RD), lambda i: (i, 0)),
            pl.BlockSpec((blk, D), lambda i: (i, 0)),
            pl.BlockSpec((blk, 1), lambda i: (i, 0)),
            pl.BlockSpec((RD, 8), lambda i: (0, 0)),
            pl.BlockSpec((RD, D), lambda i: (0, 0)),
            pl.BlockSpec((D, D), lambda i: (0, 0)),
            pl.BlockSpec((D, D), lambda i: (0, 0)),
        ],
        out_specs=pl.BlockSpec((blk, D), lambda i: (i, 0)),
        out_shape=jax.ShapeDtypeStruct((E, D), jnp.float32),
    )(rev, hsrc, cjs2, wpr, w1, w2, w3)


@functools.partial(
    pl.kernel,
    mesh=_mesh,
    compiler_params=_sc_cp,
    out_type=jax.ShapeDtypeStruct((N, D), jnp.float32),
    scratch_types=[
        pltpu.VMEM((ACCR, D), jnp.float32),
        pltpu.VMEM((SCHUNK,), jnp.int32),
        pltpu.VMEM((SCHUNK,), jnp.int32),
        pltpu.VMEM((LMAX + 64,), jnp.int32),
        pltpu.VMEM((BATCH,), jnp.int32),
        pltpu.VMEM((BATCH,), jnp.int32),
        pltpu.VMEM((BATCH,), jnp.int32),
        pltpu.VMEM((BATCH,), jnp.int32),
        pltpu.VMEM((BATCH, D), jnp.float32),
        pltpu.VMEM((BATCH, D), jnp.float32),
        pltpu.SemaphoreType.DMA,
        pltpu.SemaphoreType.DMA,
        pltpu.SemaphoreType.DMA,
        pltpu.SemaphoreType.DMA,
    ],
)
def _sc_scatter(m_hbm, dst_hbm, zero_hbm, out_hbm,
                acc_v, dstc0, dstc1, list_v, eid0, ldst0, eid1, ldst1,
                rows0, rows1, semr0, semr1, semd0, semd1):
    wid = lax.axis_index("s") * NC + lax.axis_index("c")
    node_base = wid * RANGE
    iota = lax.iota(jnp.int32, 16)

    pltpu.sync_copy(zero_hbm, acc_v.at[pl.ds(0, 128)])
    pltpu.sync_copy(zero_hbm, acc_v.at[pl.ds(128, 128)])
    pltpu.sync_copy(zero_hbm.at[pl.ds(0, ACCR - 256)], acc_v.at[pl.ds(256, ACCR - 256)])

    def unpack(b, eid_r, ldst_r):
        flo = b * BATCH
        for g in range(BATCH // 16):
            sl = pl.ds(g * 16, 16)
            p = list_v[pl.ds(flo + g * 16, 16)]
            eid_r[sl] = lax.shift_right_logical(p, 9)
            ldst_r[sl] = p & 511

    cols = [iota + j * 16 for j in range(D // 16)]

    def accum(ldst_r, rows_r):
        def edge_body(i2, _):
            for u in range(2):
                i = i2 * 2 + u
                i_splat = jnp.zeros((16,), jnp.int32) + i
                r_splat = plsc.load_gather(ldst_r, [i_splat])
                for j in range(D // 16):
                    v = rows_r[i, pl.ds(j * 16, 16)]
                    plsc.addupdate_scatter(acc_v, [r_splat, cols[j]], v)
            return 0

        lax.fori_loop(0, BATCH // 2, edge_body, 0)

    def drain(nb):
        @pl.when(nb > 0)
        def _():
            unpack(0, eid0, ldst0)
            pltpu.async_copy(m_hbm.at[eid0], rows0, semr0)

        def pair_body(p, _):
            b1 = 2 * p + 1

            @pl.when(b1 < nb)
            def _():
                unpack(b1, eid1, ldst1)
                pltpu.async_copy(m_hbm.at[eid1], rows1, semr1)

            pltpu.make_async_copy(m_hbm.at[eid0], rows0, semr0).wait()
            accum(ldst0, rows0)

            @pl.when(b1 + 1 < nb)
            def _():
                unpack(b1 + 1, eid0, ldst0)
                pltpu.async_copy(m_hbm.at[eid0], rows0, semr0)

            @pl.when(b1 < nb)
            def _():
                pltpu.make_async_copy(m_hbm.at[eid1], rows1, semr1).wait()
                accum(ldst1, rows1)

            return 0

        lax.fori_loop(0, (nb + 1) >> 1, pair_body, 0)

    def scan_groups(c, dstc_v, cnt):
        def group_body(j, cnt):
            d = dstc_v[pl.ds(j * 16, 16)]
            rel = d - node_base
            ok = (rel >= 0) & (rel < RANGE)
            eid = (c * SCHUNK + j * 16) + iota
            packed = lax.shift_left(eid, 9) + rel
            plsc.store_compressed(list_v.at[pl.ds(cnt, 16)], packed, mask=ok)
            return cnt + jnp.sum(ok.astype(jnp.int32))

        cnt = lax.fori_loop(0, SCHUNK // 16, group_body, cnt)
        nb = jnp.where(cnt >= LMAX - SCHUNK, cnt >> 6, 0)
        drain(nb)
        for g in range(4):
            v = list_v[pl.ds(nb * BATCH + g * 16, 16)]
            list_v[pl.ds(g * 16, 16)] = v
        return cnt - nb * BATCH

    pltpu.async_copy(dst_hbm.at[pl.ds(0, SCHUNK)], dstc0, semd0)

    def chunk_pair(p, cnt):
        c0 = 2 * p
        pltpu.async_copy(
            dst_hbm.at[pl.ds((c0 + 1) * SCHUNK, SCHUNK)], dstc1, semd1)
        pltpu.make_async_copy(dst_hbm.at[pl.ds(0, SCHUNK)], dstc0, semd0).wait()
        cnt = scan_groups(c0, dstc0, cnt)

        @pl.when(c0 + 2 < NSCAN)
        def _():
            pltpu.async_copy(
                dst_hbm.at[pl.ds((c0 + 2) * SCHUNK, SCHUNK)], dstc0, semd0)

        pltpu.make_async_copy(dst_hbm.at[pl.ds(0, SCHUNK)], dstc1, semd1).wait()
        cnt = scan_groups(c0 + 1, dstc1, cnt)
        return cnt

    cnt = lax.fori_loop(0, NSCAN // 2, chunk_pair, 0)

    for g in range(BATCH // 16):
        list_v[pl.ds(cnt + g * 16, 16)] = jnp.full((16,), TRASH, jnp.int32)
    drain((cnt + BATCH) >> 6)

    @pl.when(wid < NW - 1)
    def _():
        pltpu.sync_copy(acc_v.at[pl.ds(0, 128)], out_hbm.at[pl.ds(node_base, 128)])
        pltpu.sync_copy(acc_v.at[pl.ds(128, 128)], out_hbm.at[pl.ds(node_base + 128, 128)])
        pltpu.sync_copy(acc_v.at[pl.ds(256, 64)], out_hbm.at[pl.ds(node_base + 256, 64)])

    @pl.when(wid == NW - 1)
    def _():
        pltpu.sync_copy(acc_v.at[pl.ds(0, 80)], out_hbm.at[pl.ds(node_base, 80)])


def _final_body(hd_ref, ci_ref, w_ref, b_ref, out_ref):
    x = hd_ref[...] * ci_ref[...]
    out_ref[...] = jnp.dot(x, w_ref[...], preferred_element_type=jnp.float32) + b_ref[...]


def _tc_final(h_dst, ci, w_lin, b_lin2):
    blk = 512
    grid = (N + blk - 1) // blk
    return pl.pallas_call(
        _final_body,
        grid=(grid,),
        in_specs=[
            pl.BlockSpec((blk, D), lambda i: (i, 0)),
            pl.BlockSpec((blk, 1), lambda i: (i, 0)),
            pl.BlockSpec((D, D), lambda i: (0, 0)),
            pl.BlockSpec((1, D), lambda i: (0, 0)),
        ],
        out_specs=pl.BlockSpec((blk, D), lambda i: (i, 0)),
        out_shape=jax.ShapeDtypeStruct((N, D), jnp.float32),
    )(h_dst, ci, w_lin, b_lin2)


def kernel(feat, edge_index, review_id, cj, ci, review_table,
           W_map, b_map, W_prob, W_rscore, W_r1, W_r2, W_r3, W_lin, b_lin):
    src = edge_index[0]
    dst = edge_index[1]
    b_map2 = b_map.reshape(1, D)
    b_lin2 = b_lin.reshape(1, D)
    wpr = jnp.concatenate(
        [W_prob, W_rscore, jnp.zeros((RD, 6), jnp.float32)], axis=1)
    zero_blk = jnp.zeros((128, D), jnp.float32)

    h = _tc_hmap(feat, W_map, b_map2)
    rev, hsrc, cj_src = _sc_gather(review_table, review_id, src, h,
                                   cj.reshape(N))
    m = _tc_mlp(rev, hsrc, cj_src.reshape(E, 1), wpr, W_r1, W_r2, W_r3)
    h_dst = _sc_scatter(m, dst, zero_blk)
    return _tc_final(h_dst, ci, W_lin, b_lin2)


# speedup vs baseline: 3.7070x; 1.0409x over previous
import dataclasses
import functools

import jax
import jax.numpy as jnp
from jax import lax
from jax.experimental import pallas as pl
from jax.experimental.pallas import tpu as pltpu
from jax.experimental.pallas import tpu_sc as plsc

N = 10000
E = 160000
D = 256
RD = 128

NC = 2
NS = 16
NW = NC * NS

GC = 128
NCHUNK = E // GC
RANGE = 320
ACCR = RANGE + 8
TRASH = RANGE + 4
SCHUNK = 2000
NSCAN = E // SCHUNK
LMAX = 6144
BATCH = 64

_mesh = plsc.VectorSubcoreMesh(core_axis_name="c", subcore_axis_name="s")

_sc_cp = pltpu.CompilerParams()
if "needs_layout_passes" in pltpu.CompilerParams.__dataclass_fields__:
    _sc_cp = dataclasses.replace(_sc_cp, needs_layout_passes=False)


def _hmap_body(feat_ref, w_ref, b_ref, out_ref):
    h = jnp.dot(feat_ref[...], w_ref[...], preferred_element_type=jnp.float32)
    out_ref[...] = h + b_ref[...]


def _tc_hmap(feat, w_map, b_map2):
    blk = 512
    grid = (N + blk - 1) // blk
    return pl.pallas_call(
        _hmap_body,
        grid=(grid,),
        in_specs=[
            pl.BlockSpec((blk, D), lambda i: (i, 0)),
            pl.BlockSpec((D, D), lambda i: (0, 0)),
            pl.BlockSpec((1, D), lambda i: (0, 0)),
        ],
        out_specs=pl.BlockSpec((blk, D), lambda i: (i, 0)),
        out_shape=jax.ShapeDtypeStruct((N, D), jnp.float32),
    )(feat, w_map, b_map2)


_GT_FULL = NCHUNK // NW


@functools.partial(
    pl.kernel,
    mesh=_mesh,
    compiler_params=_sc_cp,
    out_type=(
        jax.ShapeDtypeStruct((E, RD), jnp.float32),
        jax.ShapeDtypeStruct((E, D), jnp.float32),
        jax.ShapeDtypeStruct((E,), jnp.float32),
    ),
    scratch_types=[
        pltpu.VMEM((GC,), jnp.int32),
        pltpu.VMEM((GC,), jnp.int32),
        pltpu.VMEM((GC, RD), jnp.float32),
        pltpu.VMEM((GC, D), jnp.float32),
        pltpu.VMEM((GC,), jnp.int32),
        pltpu.VMEM((GC,), jnp.int32),
        pltpu.VMEM((GC, RD), jnp.float32),
        pltpu.VMEM((GC, D), jnp.float32),
        pltpu.VMEM((GC,), jnp.float32),
        pltpu.VMEM((N,), jnp.float32),
        pltpu.SemaphoreType.DMA,
        pltpu.SemaphoreType.DMA,
        pltpu.SemaphoreType.DMA,
        pltpu.SemaphoreType.DMA,
        pltpu.SemaphoreType.DMA,
        pltpu.SemaphoreType.DMA,
        pltpu.SemaphoreType.DMA,
        pltpu.SemaphoreType.DMA,
    ],
)
def _sc_gather(table_hbm, rid_hbm, src_hbm, h_hbm, cj_hbm,
               rev_out, hsrc_out, cjsrc_out,
               rid0, src0, rev0, hsrc0, rid1, src1, rev1, hsrc1,
               cjsrc_v, cj_v,
               sgr0, sgh0, ssr0, ssh0, sgr1, sgh1, ssr1, ssh1):
    wid = lax.axis_index("s") * NC + lax.axis_index("c")
    pltpu.sync_copy(cj_hbm, cj_v)

    sets = ((rid0, src0, rev0, hsrc0, sgr0, sgh0, ssr0, ssh0),
            (rid1, src1, rev1, hsrc1, sgr1, sgh1, ssr1, ssh1))

    def chunk_of(t):
        return t * NW + wid

    def load_and_issue(t, s):
        rid, srcv, rev, hsrc, sgr, sgh, _, _ = sets[s]
        base = chunk_of(t) * GC
        pltpu.sync_copy(rid_hbm.at[pl.ds(base, GC)], rid)
        pltpu.sync_copy(src_hbm.at[pl.ds(base, GC)], srcv)
        pltpu.async_copy(table_hbm.at[rid], rev, sgr)
        pltpu.async_copy(h_hbm.at[srcv], hsrc, sgh)

    def finish(t, s):
        rid, srcv, rev, hsrc, sgr, sgh, ssr, ssh = sets[s]
        base = chunk_of(t) * GC
        pltpu.make_async_copy(table_hbm.at[rid], rev, sgr).wait()
        pltpu.async_copy(rev, rev_out.at[pl.ds(base, GC)], ssr)
        pltpu.make_async_copy(h_hbm.at[srcv], hsrc, sgh).wait()
        pltpu.async_copy(hsrc, hsrc_out.at[pl.ds(base, GC)], ssh)
        for j in range(GC // 16):
            sl = pl.ds(j * 16, 16)
            cjsrc_v[sl] = plsc.load_gather(cj_v, [srcv[sl]])
        pltpu.sync_copy(cjsrc_v, cjsrc_out.at[pl.ds(base, GC)])

    def wait_stores(s):
        _, _, rev, hsrc, _, _, ssr, ssh = sets[s]
        pltpu.make_async_copy(rev, rev_out.at[pl.ds(0, GC)], ssr).wait()
        pltpu.make_async_copy(hsrc, hsrc_out.at[pl.ds(0, GC)], ssh).wait()

    load_and_issue(0, 0)

    @pl.loop(0, _GT_FULL // 2 + 1)
    def _(p):
        t0 = 2 * p
        t1 = t0 + 1

        @pl.when(jnp.logical_or(t1 < _GT_FULL,
                                chunk_of(_GT_FULL) < NCHUNK))
        def _():
            @pl.when(t1 > 1)
            def _():
                wait_stores(1)

            load_and_issue(t1, 1)

        finish(t0, 0)

        @pl.when(t0 + 2 < _GT_FULL)
        def _():
            wait_stores(0)
            load_and_issue(t0 + 2, 0)

        @pl.when(jnp.logical_or(t1 < _GT_FULL,
                                chunk_of(_GT_FULL) < NCHUNK))
        def _():
            finish(t1, 1)

    wait_stores(0)
    wait_stores(1)


def _mlp_body(rev_ref, hsrc_ref, cjs_ref, wpr_ref, w1_ref, w2_ref, w3_ref, out_ref):
    bf = jnp.bfloat16
    rv = rev_ref[...].astype(bf)
    pr = jnp.dot(rv, wpr_ref[...].astype(bf), preferred_element_type=jnp.float32)
    pr = jax.nn.sigmoid(pr)
    pa = pr[:, 0:1]
    ra = pr[:, 1:2]
    g = lambda x: 0.5 * x * (1.0 + lax.erf(x * 0.7071067811865476))
    rf = g(jnp.dot(rv, w1_ref[...].astype(bf), preferred_element_type=jnp.float32))
    rf = g(jnp.dot(rf.astype(bf), w2_ref[...].astype(bf), preferred_element_type=jnp.float32))
    rf = jnp.dot(rf.astype(bf), w3_ref[...].astype(bf), preferred_element_type=jnp.float32)
    cjs = cjs_ref[...]
    out_ref[...] = hsrc_ref[...] * (pa * cjs) + rf * (ra * cjs)


def _tc_mlp(rev, hsrc, cjs2, wpr, w1, w2, w3):
    blk = 1280
    grid = E // blk
    return pl.pallas_call(
        _mlp_body,
        grid=(grid,),
        in_specs=[
            pl.BlockSpec((blk, RD), lambda i: (i, 0)),
            pl.BlockSpec((blk, D), lambda i: (i, 0)),
            pl.BlockSpec((blk, 1), lambda i: (i, 0)),
            pl.BlockSpec((RD, 8), lambda i: (0, 0)),
            pl.BlockSpec((RD, D), lambda i: (0, 0)),
            pl.BlockSpec((D, D), lambda i: (0, 0)),
            pl.BlockSpec((D, D), lambda i: (0, 0)),
        ],
        out_specs=pl.BlockSpec((blk, D), lambda i: (i, 0)),
        out_shape=jax.ShapeDtypeStruct((E, D), jnp.float32),
    )(rev, hsrc, cjs2, wpr, w1, w2, w3)


@functools.partial(
    pl.kernel,
    mesh=_mesh,
    compiler_params=_sc_cp,
    out_type=jax.ShapeDtypeStruct((N, D), jnp.float32),
    scratch_types=[
        pltpu.VMEM((ACCR, D), jnp.float32),
        pltpu.VMEM((SCHUNK,), jnp.int32),
        pltpu.VMEM((SCHUNK,), jnp.int32),
        pltpu.VMEM((LMAX + 64,), jnp.int32),
        pltpu.VMEM((BATCH,), jnp.int32),
        pltpu.VMEM((BATCH,), jnp.int32),
        pltpu.VMEM((BATCH,), jnp.int32),
        pltpu.VMEM((BATCH,), jnp.int32),
        pltpu.VMEM((BATCH, D), jnp.float32),
        pltpu.VMEM((BATCH, D), jnp.float32),
        pltpu.SemaphoreType.DMA,
        pltpu.SemaphoreType.DMA,
        pltpu.SemaphoreType.DMA,
        pltpu.SemaphoreType.DMA,
    ],
)
def _sc_scatter(m_hbm, dst_hbm, zero_hbm, out_hbm,
                acc_v, dstc0, dstc1, list_v, eid0, ldst0, eid1, ldst1,
                rows0, rows1, semr0, semr1, semd0, semd1):
    wid = lax.axis_index("s") * NC + lax.axis_index("c")
    node_base = wid * RANGE
    iota = lax.iota(jnp.int32, 16)

    pltpu.sync_copy(zero_hbm, acc_v.at[pl.ds(0, 128)])
    pltpu.sync_copy(zero_hbm, acc_v.at[pl.ds(128, 128)])
    pltpu.sync_copy(zero_hbm.at[pl.ds(0, ACCR - 256)], acc_v.at[pl.ds(256, ACCR - 256)])

    def unpack(b, eid_r, ldst_r):
        flo = b * BATCH
        for g in range(BATCH // 16):
            sl = pl.ds(g * 16, 16)
            p = list_v[pl.ds(flo + g * 16, 16)]
            eid_r[sl] = lax.shift_right_logical(p, 9)
            ldst_r[sl] = p & 511

    cols = [iota + j * 16 for j in range(D // 16)]

    def accum(ldst_r, rows_r):
        def edge_body(i2, _):
            for u in range(2):
                i = i2 * 2 + u
                i_splat = jnp.zeros((16,), jnp.int32) + i
                r_splat = plsc.load_gather(ldst_r, [i_splat])
                for j in range(D // 16):
                    v = rows_r[i, pl.ds(j * 16, 16)]
                    plsc.addupdate_scatter(acc_v, [r_splat, cols[j]], v)
            return 0

        lax.fori_loop(0, BATCH // 2, edge_body, 0)

    def drain(nb):
        @pl.when(nb > 0)
        def _():
            unpack(0, eid0, ldst0)
            pltpu.async_copy(m_hbm.at[eid0], rows0, semr0)

        def pair_body(p, _):
            b1 = 2 * p + 1

            @pl.when(b1 < nb)
            def _():
                unpack(b1, eid1, ldst1)
                pltpu.async_copy(m_hbm.at[eid1], rows1, semr1)

            pltpu.make_async_copy(m_hbm.at[eid0], rows0, semr0).wait()
            accum(ldst0, rows0)

            @pl.when(b1 + 1 < nb)
            def _():
                unpack(b1 + 1, eid0, ldst0)
                pltpu.async_copy(m_hbm.at[eid0], rows0, semr0)

            @pl.when(b1 < nb)
            def _():
                pltpu.make_async_copy(m_hbm.at[eid1], rows1, semr1).wait()
                accum(ldst1, rows1)

            return 0

        lax.fori_loop(0, (nb + 1) >> 1, pair_body, 0)

    def scan_groups(c, dstc_v, cnt):
        def group_body(j, cnt):
            d = dstc_v[pl.ds(j * 16, 16)]
            rel = d - node_base
            ok = (rel >= 0) & (rel < RANGE)
            eid = (c * SCHUNK + j * 16) + iota
            packed = lax.shift_left(eid, 9) + rel
            plsc.store_compressed(list_v.at[pl.ds(cnt, 16)], packed, mask=ok)
            return cnt + jnp.sum(ok.astype(jnp.int32))

        cnt = lax.fori_loop(0, SCHUNK // 16, group_body, cnt)
        nb = jnp.where(cnt >= LMAX - SCHUNK, cnt >> 6, 0)
        drain(nb)
        for g in range(4):
            v = list_v[pl.ds(nb * BATCH + g * 16, 16)]
            list_v[pl.ds(g * 16, 16)] = v
        return cnt - nb * BATCH

    pltpu.async_copy(dst_hbm.at[pl.ds(0, SCHUNK)], dstc0, semd0)

    def chunk_pair(p, cnt):
        c0 = 2 * p
        pltpu.async_copy(
            dst_hbm.at[pl.ds((c0 + 1) * SCHUNK, SCHUNK)], dstc1, semd1)
        pltpu.make_async_copy(dst_hbm.at[pl.ds(0, SCHUNK)], dstc0, semd0).wait()
        cnt = scan_groups(c0, dstc0, cnt)

        @pl.when(c0 + 2 < NSCAN)
        def _():
            pltpu.async_copy(
                dst_hbm.at[pl.ds((c0 + 2) * SCHUNK, SCHUNK)], dstc0, semd0)

        pltpu.make_async_copy(dst_hbm.at[pl.ds(0, SCHUNK)], dstc1, semd1).wait()
        cnt = scan_groups(c0 + 1, dstc1, cnt)
        return cnt

    cnt = lax.fori_loop(0, NSCAN // 2, chunk_pair, 0)

    for g in range(BATCH // 16):
        list_v[pl.ds(cnt + g * 16, 16)] = jnp.full((16,), TRASH, jnp.int32)
    drain((cnt + BATCH) >> 6)

    @pl.when(wid < NW - 1)
    def _():
        pltpu.sync_copy(acc_v.at[pl.ds(0, 128)], out_hbm.at[pl.ds(node_base, 128)])
        pltpu.sync_copy(acc_v.at[pl.ds(128, 128)], out_hbm.at[pl.ds(node_base + 128, 128)])
        pltpu.sync_copy(acc_v.at[pl.ds(256, 64)], out_hbm.at[pl.ds(node_base + 256, 64)])

    @pl.when(wid == NW - 1)
    def _():
        pltpu.sync_copy(acc_v.at[pl.ds(0, 80)], out_hbm.at[pl.ds(node_base, 80)])


def _final_body(hd_ref, ci_ref, w_ref, b_ref, out_ref):
    x = hd_ref[...] * ci_ref[...]
    out_ref[...] = jnp.dot(x, w_ref[...], preferred_element_type=jnp.float32) + b_ref[...]


def _tc_final(h_dst, ci, w_lin, b_lin2):
    blk = 512
    grid = (N + blk - 1) // blk
    return pl.pallas_call(
        _final_body,
        grid=(grid,),
        in_specs=[
            pl.BlockSpec((blk, D), lambda i: (i, 0)),
            pl.BlockSpec((blk, 1), lambda i: (i, 0)),
            pl.BlockSpec((D, D), lambda i: (0, 0)),
            pl.BlockSpec((1, D), lambda i: (0, 0)),
        ],
        out_specs=pl.BlockSpec((blk, D), lambda i: (i, 0)),
        out_shape=jax.ShapeDtypeStruct((N, D), jnp.float32),
    )(h_dst, ci, w_lin, b_lin2)


def kernel(feat, edge_index, review_id, cj, ci, review_table,
           W_map, b_map, W_prob, W_rscore, W_r1, W_r2, W_r3, W_lin, b_lin):
    src = edge_index[0]
    dst = edge_index[1]
    b_map2 = b_map.reshape(1, D)
    b_lin2 = b_lin.reshape(1, D)
    wpr = jnp.concatenate(
        [W_prob, W_rscore, jnp.zeros((RD, 6), jnp.float32)], axis=1)
    zero_blk = jnp.zeros((128, D), jnp.float32)

    h = _tc_hmap(feat, W_map, b_map2)
    rev, hsrc, cj_src = _sc_gather(review_table, review_id, src, h,
                                   cj.reshape(N))
    m = _tc_mlp(rev, hsrc, cj_src.reshape(E, 1), wpr, W_r1, W_r2, W_r3)
    h_dst = _sc_scatter(m, dst, zero_blk)
    return _tc_final(h_dst, ci, W_lin, b_lin2)


# speedup vs baseline: 4.1107x; 1.1089x over previous
import dataclasses
import functools

import jax
import jax.numpy as jnp
from jax import lax
from jax.experimental import pallas as pl
from jax.experimental.pallas import tpu as pltpu
from jax.experimental.pallas import tpu_sc as plsc

N = 10000
E = 160000
D = 256
RD = 128

NC = 2
NS = 16
NW = NC * NS

GC = 128
NCHUNK = E // GC
RANGE = 320
ACCR = RANGE + 8
TRASH = RANGE + 4
SCHUNK = 2000
NSCAN = E // SCHUNK
LMAX = 6464
FLUSH = 4096
LW = 8192
REG = 64 + E + LW + 1024
BATCH = 64

_mesh = plsc.VectorSubcoreMesh(core_axis_name="c", subcore_axis_name="s")

_sc_cp = pltpu.CompilerParams()
if "needs_layout_passes" in pltpu.CompilerParams.__dataclass_fields__:
    _sc_cp = dataclasses.replace(_sc_cp, needs_layout_passes=False)


def _hmap_body(feat_ref, w_ref, b_ref, out_ref):
    h = jnp.dot(feat_ref[...], w_ref[...], preferred_element_type=jnp.float32)
    out_ref[...] = h + b_ref[...]


def _tc_hmap(feat, w_map, b_map2):
    blk = 512
    grid = (N + blk - 1) // blk
    return pl.pallas_call(
        _hmap_body,
        grid=(grid,),
        in_specs=[
            pl.BlockSpec((blk, D), lambda i: (i, 0)),
            pl.BlockSpec((D, D), lambda i: (0, 0)),
            pl.BlockSpec((1, D), lambda i: (0, 0)),
        ],
        out_specs=pl.BlockSpec((blk, D), lambda i: (i, 0)),
        out_shape=jax.ShapeDtypeStruct((N, D), jnp.float32),
    )(feat, w_map, b_map2)


_GT_FULL = NCHUNK // NW


@functools.partial(
    pl.kernel,
    mesh=_mesh,
    compiler_params=_sc_cp,
    out_type=(
        jax.ShapeDtypeStruct((E, RD), jnp.float32),
        jax.ShapeDtypeStruct((E, D), jnp.float32),
        jax.ShapeDtypeStruct((E,), jnp.float32),
    ),
    scratch_types=[
        pltpu.VMEM((GC,), jnp.int32),
        pltpu.VMEM((GC,), jnp.int32),
        pltpu.VMEM((GC, RD), jnp.float32),
        pltpu.VMEM((GC, D), jnp.float32),
        pltpu.VMEM((GC,), jnp.int32),
        pltpu.VMEM((GC,), jnp.int32),
        pltpu.VMEM((GC, RD), jnp.float32),
        pltpu.VMEM((GC, D), jnp.float32),
        pltpu.VMEM((GC,), jnp.float32),
        pltpu.VMEM((N,), jnp.float32),
        pltpu.SemaphoreType.DMA,
        pltpu.SemaphoreType.DMA,
        pltpu.SemaphoreType.DMA,
        pltpu.SemaphoreType.DMA,
        pltpu.SemaphoreType.DMA,
        pltpu.SemaphoreType.DMA,
        pltpu.SemaphoreType.DMA,
        pltpu.SemaphoreType.DMA,
    ],
)
def _sc_gather(table_hbm, rid_hbm, src_hbm, h_hbm, cj_hbm,
               rev_out, hsrc_out, cjsrc_out,
               rid0, src0, rev0, hsrc0, rid1, src1, rev1, hsrc1,
               cjsrc_v, cj_v,
               sgr0, sgh0, ssr0, ssh0, sgr1, sgh1, ssr1, ssh1):
    wid = lax.axis_index("s") * NC + lax.axis_index("c")
    pltpu.sync_copy(cj_hbm, cj_v)

    sets = ((rid0, src0, rev0, hsrc0, sgr0, sgh0, ssr0, ssh0),
            (rid1, src1, rev1, hsrc1, sgr1, sgh1, ssr1, ssh1))

    def chunk_of(t):
        return t * NW + wid

    def load_and_issue(t, s):
        rid, srcv, rev, hsrc, sgr, sgh, _, _ = sets[s]
        base = chunk_of(t) * GC
        pltpu.sync_copy(rid_hbm.at[pl.ds(base, GC)], rid)
        pltpu.sync_copy(src_hbm.at[pl.ds(base, GC)], srcv)
        pltpu.async_copy(table_hbm.at[rid], rev, sgr)
        pltpu.async_copy(h_hbm.at[srcv], hsrc, sgh)

    def finish(t, s):
        rid, srcv, rev, hsrc, sgr, sgh, ssr, ssh = sets[s]
        base = chunk_of(t) * GC
        pltpu.make_async_copy(table_hbm.at[rid], rev, sgr).wait()
        pltpu.async_copy(rev, rev_out.at[pl.ds(base, GC)], ssr)
        pltpu.make_async_copy(h_hbm.at[srcv], hsrc, sgh).wait()
        pltpu.async_copy(hsrc, hsrc_out.at[pl.ds(base, GC)], ssh)
        for j in range(GC // 16):
            sl = pl.ds(j * 16, 16)
            cjsrc_v[sl] = plsc.load_gather(cj_v, [srcv[sl]])
        pltpu.sync_copy(cjsrc_v, cjsrc_out.at[pl.ds(base, GC)])

    def wait_stores(s):
        _, _, rev, hsrc, _, _, ssr, ssh = sets[s]
        pltpu.make_async_copy(rev, rev_out.at[pl.ds(0, GC)], ssr).wait()
        pltpu.make_async_copy(hsrc, hsrc_out.at[pl.ds(0, GC)], ssh).wait()

    load_and_issue(0, 0)

    @pl.loop(0, _GT_FULL // 2 + 1)
    def _(p):
        t0 = 2 * p
        t1 = t0 + 1

        @pl.when(jnp.logical_or(t1 < _GT_FULL,
                                chunk_of(_GT_FULL) < NCHUNK))
        def _():
            @pl.when(t1 > 1)
            def _():
                wait_stores(1)

            load_and_issue(t1, 1)

        finish(t0, 0)

        @pl.when(t0 + 2 < _GT_FULL)
        def _():
            wait_stores(0)
            load_and_issue(t0 + 2, 0)

        @pl.when(jnp.logical_or(t1 < _GT_FULL,
                                chunk_of(_GT_FULL) < NCHUNK))
        def _():
            finish(t1, 1)

    wait_stores(0)
    wait_stores(1)


def _mlp_body(rev_ref, hsrc_ref, cjs_ref, wpr_ref, w1_ref, w2_ref, w3_ref, out_ref):
    bf = jnp.bfloat16
    rv = rev_ref[...].astype(bf)
    pr = jnp.dot(rv, wpr_ref[...].astype(bf), preferred_element_type=jnp.float32)
    pr = jax.nn.sigmoid(pr)
    pa = pr[:, 0:1]
    ra = pr[:, 1:2]
    g = lambda x: 0.5 * x * (1.0 + lax.erf(x * 0.7071067811865476))
    rf = g(jnp.dot(rv, w1_ref[...].astype(bf), preferred_element_type=jnp.float32))
    rf = g(jnp.dot(rf.astype(bf), w2_ref[...].astype(bf), preferred_element_type=jnp.float32))
    rf = jnp.dot(rf.astype(bf), w3_ref[...].astype(bf), preferred_element_type=jnp.float32)
    cjs = cjs_ref[...]
    out_ref[...] = hsrc_ref[...] * (pa * cjs) + rf * (ra * cjs)


def _tc_mlp(rev, hsrc, cjs2, wpr, w1, w2, w3):
    blk = 1280
    grid = E // blk
    return pl.pallas_call(
        _mlp_body,
        grid=(grid,),
        in_specs=[
            pl.BlockSpec((blk, RD), lambda i: (i, 0)),
            pl.BlockSpec((blk, D), lambda i: (i, 0)),
            pl.BlockSpec((blk, 1), lambda i: (i, 0)),
            pl.BlockSpec((RD, 8), lambda i: (0, 0)),
            pl.BlockSpec((RD, D), lambda i: (0, 0)),
            pl.BlockSpec((D, D), lambda i: (0, 0)),
            pl.BlockSpec((D, D), lambda i: (0, 0)),
        ],
        out_specs=pl.BlockSpec((blk, D), lambda i: (i, 0)),
        out_shape=jax.ShapeDtypeStruct((E, D), jnp.float32),
    )(rev, hsrc, cjs2, wpr, w1, w2, w3)


@functools.partial(
    pl.kernel,
    mesh=_mesh,
    compiler_params=_sc_cp,
    out_type=jax.ShapeDtypeStruct((NW * REG,), jnp.int32),
    scratch_types=[
        pltpu.VMEM((SCHUNK,), jnp.int32),
        pltpu.VMEM((SCHUNK,), jnp.int32),
        pltpu.VMEM((LMAX + 64,), jnp.int32),
        pltpu.VMEM((16,), jnp.int32),
        pltpu.SemaphoreType.DMA,
        pltpu.SemaphoreType.DMA,
    ],
)
def _sc_scan(dst_hbm, lists_hbm, dstc0, dstc1, list_v, hdr_v, semd0, semd1):
    wid = lax.axis_index("s") * NC + lax.axis_index("c")
    node_base = wid * RANGE
    reg_base = pl.multiple_of(wid * REG, 64)
    iota = lax.iota(jnp.int32, 16)

    def scan_groups(c, dstc_v, carry):
        cnt, hout = carry

        def group_body(j, cnt):
            d = dstc_v[pl.ds(j * 16, 16)]
            rel = d - node_base
            ok = plsc.bitcast(rel, jnp.uint32) < jnp.uint32(RANGE)
            eid = (c * SCHUNK + j * 16) + iota
            packed = lax.shift_left(eid, 9) + rel
            plsc.store_compressed(list_v.at[pl.ds(cnt, 16)], packed, mask=ok)
            return cnt + jnp.sum(ok.astype(jnp.int32))

        cnt = lax.fori_loop(0, SCHUNK // 16, group_body, cnt)
        full = (cnt >= FLUSH).astype(jnp.int32)

        @pl.when(full == 1)
        def _():
            pltpu.sync_copy(
                list_v.at[pl.ds(0, FLUSH)],
                lists_hbm.at[pl.ds(reg_base + 64 + pl.multiple_of(hout, 64),
                                   FLUSH)])

        off = full * FLUSH
        for g in range((LMAX - FLUSH) // 16):
            v = list_v[pl.ds(off + g * 16, 16)]
            list_v[pl.ds(g * 16, 16)] = v
        return cnt - off, hout + off

    pltpu.async_copy(dst_hbm.at[pl.ds(0, SCHUNK)], dstc0, semd0)

    def chunk_pair(p, carry):
        c0 = 2 * p
        pltpu.async_copy(
            dst_hbm.at[pl.ds((c0 + 1) * SCHUNK, SCHUNK)], dstc1, semd1)
        pltpu.make_async_copy(dst_hbm.at[pl.ds(0, SCHUNK)], dstc0, semd0).wait()
        carry = scan_groups(c0, dstc0, carry)

        @pl.when(c0 + 2 < NSCAN)
        def _():
            pltpu.async_copy(
                dst_hbm.at[pl.ds((c0 + 2) * SCHUNK, SCHUNK)], dstc0, semd0)

        pltpu.make_async_copy(dst_hbm.at[pl.ds(0, SCHUNK)], dstc1, semd1).wait()
        return scan_groups(c0 + 1, dstc1, carry)

    cnt, hout = lax.fori_loop(0, NSCAN // 2, chunk_pair, (0, 0))

    for g in range(BATCH // 16):
        list_v[pl.ds(cnt + g * 16, 16)] = jnp.full((16,), TRASH, jnp.int32)
    ntail = (cnt + 1023) >> 10

    hout_a = pl.multiple_of(hout, 64)

    def tail_body(b, _):
        pltpu.sync_copy(
            list_v.at[pl.ds(b * 1024, 1024)],
            lists_hbm.at[pl.ds(reg_base + 64 + hout_a + b * 1024, 1024)])
        return 0

    lax.fori_loop(0, ntail, tail_body, 0)

    total = hout + ((cnt + 63) & ~63)
    hdr_v[pl.ds(0, 16)] = jnp.zeros((16,), jnp.int32) + total
    pltpu.sync_copy(hdr_v, lists_hbm.at[pl.ds(reg_base, 16)])


@functools.partial(
    pl.kernel,
    mesh=_mesh,
    compiler_params=_sc_cp,
    out_type=jax.ShapeDtypeStruct((N, D), jnp.float32),
    scratch_types=[
        pltpu.VMEM((ACCR, D), jnp.float32),
        pltpu.VMEM((LW,), jnp.int32),
        pltpu.VMEM((16,), jnp.int32),
        pltpu.VMEM((BATCH,), jnp.int32),
        pltpu.VMEM((BATCH,), jnp.int32),
        pltpu.VMEM((BATCH,), jnp.int32),
        pltpu.VMEM((BATCH,), jnp.int32),
        pltpu.VMEM((BATCH, D), jnp.float32),
        pltpu.VMEM((BATCH, D), jnp.float32),
        pltpu.SemaphoreType.DMA,
        pltpu.SemaphoreType.DMA,
    ],
)
def _sc_accum(m_hbm, lists_hbm, zero_hbm, out_hbm,
              acc_v, list_v, hdr_v, eid0, ldst0, eid1, ldst1,
              rows0, rows1, semr0, semr1):
    wid = lax.axis_index("s") * NC + lax.axis_index("c")
    node_base = wid * RANGE
    reg_base = pl.multiple_of(wid * REG, 64)
    iota = lax.iota(jnp.int32, 16)

    pltpu.sync_copy(zero_hbm, acc_v.at[pl.ds(0, 128)])
    pltpu.sync_copy(zero_hbm, acc_v.at[pl.ds(128, 128)])
    pltpu.sync_copy(zero_hbm.at[pl.ds(0, ACCR - 256)], acc_v.at[pl.ds(256, ACCR - 256)])

    pltpu.sync_copy(lists_hbm.at[pl.ds(reg_base, 16)], hdr_v)
    total = jnp.max(hdr_v[pl.ds(0, 16)])

    def unpack(b, eid_r, ldst_r):
        flo = b * BATCH
        for g in range(BATCH // 16):
            sl = pl.ds(g * 16, 16)
            p = list_v[pl.ds(flo + g * 16, 16)]
            eid_r[sl] = lax.shift_right_logical(p, 9)
            ldst_r[sl] = p & 511
    cols = [iota + j * 16 for j in range(D // 16)]

    def accum(ldst_r, rows_r):
        def edge_body(i2, _):
            for u in range(2):
                i = i2 * 2 + u
                i_splat = jnp.zeros((16,), jnp.int32) + i
                r_splat = plsc.load_gather(ldst_r, [i_splat])
                for j in range(D // 16):
                    v = rows_r[i, pl.ds(j * 16, 16)]
                    plsc.addupdate_scatter(acc_v, [r_splat, cols[j]], v)
            return 0

        lax.fori_loop(0, BATCH // 2, edge_body, 0)

    def drain(nb):
        @pl.when(nb > 0)
        def _():
            unpack(0, eid0, ldst0)
            pltpu.async_copy(m_hbm.at[eid0], rows0, semr0)

        def pair_body(p, _):
            b1 = 2 * p + 1

            @pl.when(b1 < nb)
            def _():
                unpack(b1, eid1, ldst1)
                pltpu.async_copy(m_hbm.at[eid1], rows1, semr1)

            pltpu.make_async_copy(m_hbm.at[eid0], rows0, semr0).wait()
            accum(ldst0, rows0)

            @pl.when(b1 + 1 < nb)
            def _():
                unpack(b1 + 1, eid0, ldst0)
                pltpu.async_copy(m_hbm.at[eid0], rows0, semr0)

            @pl.when(b1 < nb)
            def _():
                pltpu.make_async_copy(m_hbm.at[eid1], rows1, semr1).wait()
                accum(ldst1, rows1)

            return 0

        lax.fori_loop(0, (nb + 1) >> 1, pair_body, 0)

    nwin = (total + LW - 1) >> 13

    def win_body(w, _):
        pltpu.sync_copy(
            lists_hbm.at[pl.ds(reg_base + 64 + w * LW, LW)], list_v)
        drain(jnp.minimum(total - w * LW, LW) >> 6)
        return 0

    lax.fori_loop(0, nwin, win_body, 0)

    @pl.when(wid < NW - 1)
    def _():
        pltpu.sync_copy(acc_v.at[pl.ds(0, 128)], out_hbm.at[pl.ds(node_base, 128)])
        pltpu.sync_copy(acc_v.at[pl.ds(128, 128)], out_hbm.at[pl.ds(node_base + 128, 128)])
        pltpu.sync_copy(acc_v.at[pl.ds(256, 64)], out_hbm.at[pl.ds(node_base + 256, 64)])

    @pl.when(wid == NW - 1)
    def _():
        pltpu.sync_copy(acc_v.at[pl.ds(0, 80)], out_hbm.at[pl.ds(node_base, 80)])


def _final_body(hd_ref, ci_ref, w_ref, b_ref, out_ref):
    x = hd_ref[...] * ci_ref[...]
    out_ref[...] = jnp.dot(x, w_ref[...], preferred_element_type=jnp.float32) + b_ref[...]


def _tc_final(h_dst, ci, w_lin, b_lin2):
    blk = 512
    grid = (N + blk - 1) // blk
    return pl.pallas_call(
        _final_body,
        grid=(grid,),
        in_specs=[
            pl.BlockSpec((blk, D), lambda i: (i, 0)),
            pl.BlockSpec((blk, 1), lambda i: (i, 0)),
            pl.BlockSpec((D, D), lambda i: (0, 0)),
            pl.BlockSpec((1, D), lambda i: (0, 0)),
        ],
        out_specs=pl.BlockSpec((blk, D), lambda i: (i, 0)),
        out_shape=jax.ShapeDtypeStruct((N, D), jnp.float32),
    )(h_dst, ci, w_lin, b_lin2)


def kernel(feat, edge_index, review_id, cj, ci, review_table,
           W_map, b_map, W_prob, W_rscore, W_r1, W_r2, W_r3, W_lin, b_lin):
    src = edge_index[0]
    dst = edge_index[1]
    b_map2 = b_map.reshape(1, D)
    b_lin2 = b_lin.reshape(1, D)
    wpr = jnp.concatenate(
        [W_prob, W_rscore, jnp.zeros((RD, 6), jnp.float32)], axis=1)
    zero_blk = jnp.zeros((128, D), jnp.float32)

    h = _tc_hmap(feat, W_map, b_map2)
    lists = _sc_scan(dst)
    rev, hsrc, cj_src = _sc_gather(review_table, review_id, src, h,
                                   cj.reshape(N))
    m = _tc_mlp(rev, hsrc, cj_src.reshape(E, 1), wpr, W_r1, W_r2, W_r3)
    h_dst = _sc_accum(m, lists, zero_blk)
    return _tc_final(h_dst, ci, W_lin, b_lin2)
